# baseline (speedup 1.0000x reference)
import functools

import jax
import jax.numpy as jnp
from jax import lax
from jax.experimental import pallas as pl
from jax.experimental.pallas import tpu as pltpu
from jax.experimental.pallas import tpu_sc as plsc

B = 16
AB = 20000
V = 12000
NW = 32
NPW = 384
VP = NW * NPW
NTAIL = V - (NW - 1) * NPW
CH = 128
IDX_ROWS = NPW // CH
ROWS = B * NPW
GROWS = ROWS // CH
NCHUNK = NPW // 16


def _body(scores_hbm, deltas_hbm, anchors_hbm, idx_hbm, out_hbm,
          idxv, gs, gc0, gc1, gc2, gc3, ax0, ax1, ax2, ax3,
          dxb, dyb, dwb, dhb, sbuf, xb, yb, wb, hb, oflat, sem):
    nc = 2
    w = lax.axis_index("s") * nc + lax.axis_index("c")
    vbase = w * NPW

    for k in range(IDX_ROWS):
        pltpu.sync_copy(idx_hbm.at[pl.ds(vbase + k * CH, CH)], idxv.at[k])

    def build(c, _):
        r = c // 8
        co = (c % 8) * 16
        sl = pl.ds(co, 16)
        vch = idxv[r, sl]
        vch4 = vch * 4
        ax0[r, sl] = vch4
        ax1[r, sl] = vch4 + 1
        ax2[r, sl] = vch4 + 2
        ax3[r, sl] = vch4 + 3
        for b in range(B):
            gs[3 * b + r, sl] = vch + b * AB
            base4 = vch4 + b * (AB * 4)
            gc0[3 * b + r, sl] = base4
            gc1[3 * b + r, sl] = base4 + 1
            gc2[3 * b + r, sl] = base4 + 2
            gc3[3 * b + r, sl] = base4 + 3
        return _
    lax.fori_loop(0, NCHUNK, build, None)

    def fire(j, _):
        dsl = pl.ds(j * CH, CH)
        pltpu.async_copy(deltas_hbm.at[gc0.at[j]], dxb.at[dsl], sem)
        pltpu.async_copy(deltas_hbm.at[gc1.at[j]], dyb.at[dsl], sem)
        pltpu.async_copy(deltas_hbm.at[gc2.at[j]], dwb.at[dsl], sem)
        pltpu.async_copy(deltas_hbm.at[gc3.at[j]], dhb.at[dsl], sem)
        pltpu.async_copy(scores_hbm.at[gs.at[j]], sbuf.at[dsl], sem)
        return _
    lax.fori_loop(0, GROWS, fire, None)
    for k in range(IDX_ROWS):
        asl = pl.ds(k * CH, CH)
        pltpu.async_copy(anchors_hbm.at[ax0.at[k]], xb.at[asl], sem)
        pltpu.async_copy(anchors_hbm.at[ax1.at[k]], yb.at[asl], sem)
        pltpu.async_copy(anchors_hbm.at[ax2.at[k]], wb.at[asl], sem)
        pltpu.async_copy(anchors_hbm.at[ax3.at[k]], hb.at[asl], sem)
    for buf in (dxb, dyb, dwb, dhb, sbuf):
        pltpu.make_async_copy(scores_hbm.at[pl.ds(0, ROWS)], buf, sem).wait()
    for buf in (xb, yb, wb, hb):
        pltpu.make_async_copy(scores_hbm.at[pl.ds(0, NPW)], buf, sem).wait()

    iota = lax.iota(jnp.int32, 16)
    i5 = iota * 5

    def compute(c, _):
        asl = pl.ds(c * 16, 16)
        x = xb[asl]
        y = yb[asl]
        ww = wb[asl]
        hh = hb[asl]
        for b in range(B):
            p0 = b * NPW + c * 16
            sl = pl.ds(p0, 16)
            obase = p0 * 5 + i5
            plsc.store_scatter(oflat, [obase], sbuf[sl])
            plsc.store_scatter(oflat, [obase + 1], x + dxb[sl] * ww)
            plsc.store_scatter(oflat, [obase + 2], y + dyb[sl] * hh)
            plsc.store_scatter(oflat, [obase + 3], ww * jnp.exp(dwb[sl]))
            plsc.store_scatter(oflat, [obase + 4], hh * jnp.exp(dhb[sl]))
        return _
    lax.fori_loop(0, NCHUNK, compute, None)

    @pl.when(w < NW - 1)
    def _full():
        cps = [pltpu.make_async_copy(
                   oflat.at[pl.ds(b * NPW * 5, NPW * 5)],
                   out_hbm.at[pl.ds((b * V + vbase) * 5, NPW * 5)], sem)
               for b in range(B)]
        for cp in cps:
            cp.start()
        for cp in cps:
            cp.wait()

    @pl.when(w == NW - 1)
    def _tail():
        cps = [pltpu.make_async_copy(
                   oflat.at[pl.ds(b * NPW * 5, NTAIL * 5)],
                   out_hbm.at[pl.ds((b * V + vbase) * 5, NTAIL * 5)], sem)
               for b in range(B)]
        for cp in cps:
            cp.start()
        for cp in cps:
            cp.wait()


@jax.jit
def _run(scores_flat, deltas_flat, anchors_flat, idx_pad):
    mesh = plsc.VectorSubcoreMesh(core_axis_name="c", subcore_axis_name="s")
    f = functools.partial(
        pl.kernel,
        out_type=jax.ShapeDtypeStruct((B * V * 5,), jnp.float32),
        mesh=mesh,
        compiler_params=pltpu.CompilerParams(needs_layout_passes=False),
        scratch_types=[
            pltpu.VMEM((IDX_ROWS, CH), jnp.int32),
            pltpu.VMEM((GROWS, CH), jnp.int32),
            pltpu.VMEM((GROWS, CH), jnp.int32),
            pltpu.VMEM((GROWS, CH), jnp.int32),
            pltpu.VMEM((GROWS, CH), jnp.int32),
            pltpu.VMEM((GROWS, CH), jnp.int32),
            pltpu.VMEM((IDX_ROWS, CH), jnp.int32),
            pltpu.VMEM((IDX_ROWS, CH), jnp.int32),
            pltpu.VMEM((IDX_ROWS, CH), jnp.int32),
            pltpu.VMEM((IDX_ROWS, CH), jnp.int32),
            pltpu.VMEM((ROWS,), jnp.float32),
            pltpu.VMEM((ROWS,), jnp.float32),
            pltpu.VMEM((ROWS,), jnp.float32),
            pltpu.VMEM((ROWS,), jnp.float32),
            pltpu.VMEM((ROWS,), jnp.float32),
            pltpu.VMEM((NPW,), jnp.float32),
            pltpu.VMEM((NPW,), jnp.float32),
            pltpu.VMEM((NPW,), jnp.float32),
            pltpu.VMEM((NPW,), jnp.float32),
            pltpu.VMEM((ROWS * 5,), jnp.float32),
            pltpu.SemaphoreType.DMA,
        ],
    )(_body)
    return f(scores_flat, deltas_flat, anchors_flat, idx_pad)


def kernel(scores, deltas, anchor_boxes, valid_indices):
    vi = valid_indices.astype(jnp.int32)
    idx_pad = jnp.zeros((VP,), jnp.int32).at[:V].set(vi)
    out = _run(scores.reshape(B * AB),
               deltas.reshape(B * AB * 4),
               anchor_boxes.reshape(AB * 4),
               idx_pad)
    return out.reshape(B, V, 5)


# speedup vs baseline: 4.5784x; 4.5784x over previous
import functools

import jax
import jax.numpy as jnp
from jax import lax
from jax.experimental import pallas as pl
from jax.experimental.pallas import tpu as pltpu
from jax.experimental.pallas import tpu_sc as plsc

B = 16
AB = 20000
V = 12000
NW = 32
NPW = 384
VP = NW * NPW
NTAIL = V - (NW - 1) * NPW
CH = 128
IDX_ROWS = NPW // CH
ROWS = B * NPW
DROWS = 4 * B * IDX_ROWS
NCHUNK = NPW // 16


def _body(scores_hbm, deltas_hbm, anchors_hbm, idx_hbm, out_hbm,
          idxv, gs, gd, ax0, ax1, ax2, ax3,
          sb, db, a0b, a1b, a2b, a3b, ob, sem):
    nc = 2
    w = lax.axis_index("s") * nc + lax.axis_index("c")
    vbase = w * NPW

    for k in range(IDX_ROWS):
        pltpu.sync_copy(idx_hbm.at[pl.ds(vbase + k * CH, CH)], idxv.at[k])

    def build(c, _):
        r = c // 8
        co = (c % 8) * 16
        sl = pl.ds(co, 16)
        vch = idxv[r, sl]
        ax0[r, sl] = vch
        ax1[r, sl] = vch + AB
        ax2[r, sl] = vch + 2 * AB
        ax3[r, sl] = vch + 3 * AB
        for b in range(B):
            gs[3 * b + r, sl] = vch + b * AB
            vb = vch + b * (4 * AB)
            gd[(4 * b) * 3 + r, sl] = vb
            gd[(4 * b + 1) * 3 + r, sl] = vb + AB
            gd[(4 * b + 2) * 3 + r, sl] = vb + 2 * AB
            gd[(4 * b + 3) * 3 + r, sl] = vb + 3 * AB
        return _
    lax.fori_loop(0, NCHUNK, build, None)

    def fire_s(j, _):
        pltpu.async_copy(scores_hbm.at[gs.at[j]],
                         sb.at[pl.ds(j * CH, CH)], sem)
        return _
    lax.fori_loop(0, B * IDX_ROWS, fire_s, None)

    def fire_d(j, _):
        pltpu.async_copy(deltas_hbm.at[gd.at[j]],
                         db.at[pl.ds(j * CH, CH)], sem)
        return _
    lax.fori_loop(0, DROWS, fire_d, None)

    for k in range(IDX_ROWS):
        ksl = pl.ds(k * CH, CH)
        pltpu.async_copy(anchors_hbm.at[ax0.at[k]], a0b.at[ksl], sem)
        pltpu.async_copy(anchors_hbm.at[ax1.at[k]], a1b.at[ksl], sem)
        pltpu.async_copy(anchors_hbm.at[ax2.at[k]], a2b.at[ksl], sem)
        pltpu.async_copy(anchors_hbm.at[ax3.at[k]], a3b.at[ksl], sem)

    pltpu.make_async_copy(scores_hbm.at[pl.ds(0, ROWS)], sb, sem).wait()
    pltpu.make_async_copy(deltas_hbm.at[pl.ds(0, 4 * ROWS)], db, sem).wait()
    for buf in (a0b, a1b, a2b, a3b):
        pltpu.make_async_copy(anchors_hbm.at[pl.ds(0, NPW)], buf, sem).wait()

    def compute(c, _):
        asl = pl.ds(c * 16, 16)
        x = a0b[asl]
        y = a1b[asl]
        ww = a2b[asl]
        hh = a3b[asl]
        for b in range(B):
            p0 = b * NPW + c * 16
            sl = pl.ds(p0, 16)
            d0 = db[pl.ds(b * (4 * NPW) + c * 16, 16)]
            d1 = db[pl.ds(b * (4 * NPW) + NPW + c * 16, 16)]
            d2 = db[pl.ds(b * (4 * NPW) + 2 * NPW + c * 16, 16)]
            d3 = db[pl.ds(b * (4 * NPW) + 3 * NPW + c * 16, 16)]
            ob[sl] = sb[sl]
            ob[pl.ds(ROWS + p0, 16)] = x + d0 * ww
            ob[pl.ds(2 * ROWS + p0, 16)] = y + d1 * hh
            ob[pl.ds(3 * ROWS + p0, 16)] = ww * jnp.exp(d2)
            ob[pl.ds(4 * ROWS + p0, 16)] = hh * jnp.exp(d3)
        return _
    lax.fori_loop(0, NCHUNK, compute, None)

    @pl.when(w < NW - 1)
    def _full():
        def wb(j, _):
            pltpu.async_copy(ob.at[pl.ds(j * NPW, NPW)],
                             out_hbm.at[pl.ds(j * V + vbase, NPW)], sem)
            return _
        lax.fori_loop(0, 5 * B, wb, None)
        pltpu.make_async_copy(out_hbm.at[pl.ds(0, 5 * ROWS)], ob, sem).wait()

    @pl.when(w == NW - 1)
    def _tail():
        def wb(j, _):
            pltpu.async_copy(ob.at[pl.ds(j * NPW, NTAIL)],
                             out_hbm.at[pl.ds(j * V + vbase, NTAIL)], sem)
            return _
        lax.fori_loop(0, 5 * B, wb, None)
        for j in range(5 * B):
            pltpu.make_async_copy(out_hbm.at[pl.ds(0, NTAIL)],
                                  ob.at[pl.ds(j * NPW, NTAIL)], sem).wait()


@jax.jit
def _run(scores_flat, deltas_flat, anchors_flat, idx_pad):
    mesh = plsc.VectorSubcoreMesh(core_axis_name="c", subcore_axis_name="s")
    f = functools.partial(
        pl.kernel,
        out_type=jax.ShapeDtypeStruct((5 * B * V,), jnp.float32),
        mesh=mesh,
        compiler_params=pltpu.CompilerParams(needs_layout_passes=False),
        scratch_types=[
            pltpu.VMEM((IDX_ROWS, CH), jnp.int32),
            pltpu.VMEM((B * IDX_ROWS, CH), jnp.int32),
            pltpu.VMEM((DROWS, CH), jnp.int32),
            pltpu.VMEM((IDX_ROWS, CH), jnp.int32),
            pltpu.VMEM((IDX_ROWS, CH), jnp.int32),
            pltpu.VMEM((IDX_ROWS, CH), jnp.int32),
            pltpu.VMEM((IDX_ROWS, CH), jnp.int32),
            pltpu.VMEM((ROWS,), jnp.float32),
            pltpu.VMEM((4 * ROWS,), jnp.float32),
            pltpu.VMEM((NPW,), jnp.float32),
            pltpu.VMEM((NPW,), jnp.float32),
            pltpu.VMEM((NPW,), jnp.float32),
            pltpu.VMEM((NPW,), jnp.float32),
            pltpu.VMEM((5 * ROWS,), jnp.float32),
            pltpu.SemaphoreType.DMA,
        ],
    )(_body)
    return f(scores_flat, deltas_flat, anchors_flat, idx_pad)


def kernel(scores, deltas, anchor_boxes, valid_indices):
    vi = valid_indices.astype(jnp.int32)
    idx_pad = jnp.zeros((VP,), jnp.int32).at[:V].set(vi)
    out = _run(scores.reshape(B * AB),
               deltas.transpose(0, 2, 1).reshape(B * 4 * AB),
               anchor_boxes.T.reshape(4 * AB),
               idx_pad)
    return out.reshape(5, B, V).transpose(1, 2, 0)


# speedup vs baseline: 5.0369x; 1.1001x over previous
import functools

import jax
import jax.numpy as jnp
from jax import lax
from jax.experimental import pallas as pl
from jax.experimental.pallas import tpu as pltpu
from jax.experimental.pallas import tpu_sc as plsc

B = 16
AB = 20000
V = 12000
NW = 32
NPW = 384
VP = NW * NPW
NTAIL = V - (NW - 1) * NPW
CH = 128
IDX_ROWS = NPW // CH
ROWS = B * NPW
DROWS = 4 * B * IDX_ROWS
NCHUNK = NPW // 16


def _body(scores_hbm, deltas_hbm, anchors_hbm, idx_hbm, out_hbm,
          idxv, gs, gd, ax0, ax1, ax2, ax3,
          sb, db, a0b, a1b, a2b, a3b, ob, sem):
    nc = 2
    w = lax.axis_index("s") * nc + lax.axis_index("c")
    vbase = w * NPW

    for k in range(IDX_ROWS):
        pltpu.sync_copy(idx_hbm.at[pl.ds(vbase + k * CH, CH)], idxv.at[k])

    def build(c, _):
        r = c // 8
        co = (c % 8) * 16
        sl = pl.ds(co, 16)
        vch = idxv[r, sl]
        ax0[r, sl] = vch
        ax1[r, sl] = vch + AB
        ax2[r, sl] = vch + 2 * AB
        ax3[r, sl] = vch + 3 * AB
        for b in range(B):
            j4 = ((3 * b + r) * 4)
            gs[3 * b + r, sl] = vch + b * AB
            vb = vch + b * (4 * AB)
            gd[j4, sl] = vb
            gd[j4 + 1, sl] = vb + AB
            gd[j4 + 2, sl] = vb + 2 * AB
            gd[j4 + 3, sl] = vb + 3 * AB
        return _
    lax.fori_loop(0, NCHUNK, build, None)

    def fire(j, _):
        pltpu.async_copy(scores_hbm.at[gs.at[j]],
                         sb.at[pl.ds(j * CH, CH)], sem)
        for cc in range(4):
            pltpu.async_copy(deltas_hbm.at[gd.at[j * 4 + cc]],
                             db.at[pl.ds((j * 4 + cc) * CH, CH)], sem)
        return _
    lax.fori_loop(0, B * IDX_ROWS, fire, None)

    for k in range(IDX_ROWS):
        ksl = pl.ds(k * CH, CH)
        pltpu.async_copy(anchors_hbm.at[ax0.at[k]], a0b.at[ksl], sem)
        pltpu.async_copy(anchors_hbm.at[ax1.at[k]], a1b.at[ksl], sem)
        pltpu.async_copy(anchors_hbm.at[ax2.at[k]], a2b.at[ksl], sem)
        pltpu.async_copy(anchors_hbm.at[ax3.at[k]], a3b.at[ksl], sem)

    pltpu.make_async_copy(scores_hbm.at[pl.ds(0, ROWS)], sb, sem).wait()
    pltpu.make_async_copy(deltas_hbm.at[pl.ds(0, 4 * ROWS)], db, sem).wait()
    for buf in (a0b, a1b, a2b, a3b):
        pltpu.make_async_copy(anchors_hbm.at[pl.ds(0, NPW)], buf, sem).wait()

    def compute(c, _):
        asl = pl.ds(c * 16, 16)
        r4 = (c // 8) * 4
        co = (c % 8) * 16
        x = a0b[asl]
        y = a1b[asl]
        ww = a2b[asl]
        hh = a3b[asl]
        for b in range(B):
            p0 = b * NPW + c * 16
            sl = pl.ds(p0, 16)
            dbase = (3 * b * 4 + r4) * CH + co
            d0 = db[pl.ds(dbase, 16)]
            d1 = db[pl.ds(dbase + CH, 16)]
            d2 = db[pl.ds(dbase + 2 * CH, 16)]
            d3 = db[pl.ds(dbase + 3 * CH, 16)]
            ob[sl] = sb[sl]
            ob[pl.ds(ROWS + p0, 16)] = x + d0 * ww
            ob[pl.ds(2 * ROWS + p0, 16)] = y + d1 * hh
            ob[pl.ds(3 * ROWS + p0, 16)] = ww * jnp.exp(d2)
            ob[pl.ds(4 * ROWS + p0, 16)] = hh * jnp.exp(d3)
        return _
    lax.fori_loop(0, NCHUNK, compute, None)

    @pl.when(w < NW - 1)
    def _full():
        def wb(j, _):
            pltpu.async_copy(ob.at[pl.ds(j * NPW, NPW)],
                             out_hbm.at[pl.ds(j * V + vbase, NPW)], sem)
            return _
        lax.fori_loop(0, 5 * B, wb, None)
        pltpu.make_async_copy(out_hbm.at[pl.ds(0, 5 * ROWS)], ob, sem).wait()

    @pl.when(w == NW - 1)
    def _tail():
        def wb(j, _):
            pltpu.async_copy(ob.at[pl.ds(j * NPW, NTAIL)],
                             out_hbm.at[pl.ds(j * V + vbase, NTAIL)], sem)
            return _
        lax.fori_loop(0, 5 * B, wb, None)
        for j in range(5 * B):
            pltpu.make_async_copy(out_hbm.at[pl.ds(0, NTAIL)],
                                  ob.at[pl.ds(j * NPW, NTAIL)], sem).wait()


@jax.jit
def _run(scores_flat, deltas_flat, anchors_flat, idx_pad):
    mesh = plsc.VectorSubcoreMesh(core_axis_name="c", subcore_axis_name="s")
    f = functools.partial(
        pl.kernel,
        out_type=jax.ShapeDtypeStruct((5 * B * V,), jnp.float32),
        mesh=mesh,
        compiler_params=pltpu.CompilerParams(needs_layout_passes=False),
        scratch_types=[
            pltpu.VMEM((IDX_ROWS, CH), jnp.int32),
            pltpu.VMEM((B * IDX_ROWS, CH), jnp.int32),
            pltpu.VMEM((DROWS, CH), jnp.int32),
            pltpu.VMEM((IDX_ROWS, CH), jnp.int32),
            pltpu.VMEM((IDX_ROWS, CH), jnp.int32),
            pltpu.VMEM((IDX_ROWS, CH), jnp.int32),
            pltpu.VMEM((IDX_ROWS, CH), jnp.int32),
            pltpu.VMEM((ROWS,), jnp.float32),
            pltpu.VMEM((4 * ROWS,), jnp.float32),
            pltpu.VMEM((NPW,), jnp.float32),
            pltpu.VMEM((NPW,), jnp.float32),
            pltpu.VMEM((NPW,), jnp.float32),
            pltpu.VMEM((NPW,), jnp.float32),
            pltpu.VMEM((5 * ROWS,), jnp.float32),
            pltpu.SemaphoreType.DMA,
        ],
    )(_body)
    return f(scores_flat, deltas_flat, anchors_flat, idx_pad)


def kernel(scores, deltas, anchor_boxes, valid_indices):
    vi = valid_indices.astype(jnp.int32)
    idx_pad = jnp.zeros((VP,), jnp.int32).at[:V].set(vi)
    out = _run(scores.reshape(B * AB),
               deltas.transpose(0, 2, 1).reshape(B * 4 * AB),
               anchor_boxes.T.reshape(4 * AB),
               idx_pad)
    return out.reshape(5, B, V).transpose(1, 2, 0)
